# baseline (speedup 1.0000x reference)
import functools

import jax
import jax.numpy as jnp
from jax import lax
from jax.experimental import pallas as pl
from jax.experimental.pallas import tpu as pltpu
from jax.experimental.pallas import tpu_sc as plsc

_NUM_OBJ = 151
_NUM_RELS = 51
_BATCH = 16384

_INFO = plsc.get_sparse_core_info()
_NC = 1
_NS = _INFO.num_subcores
_NW = _NC * _NS
_L = _INFO.num_lanes
_COLS = _BATCH // _NW
_GATHER_W = 128
_NG = _COLS // _GATHER_W
_CPG = _GATHER_W // _L


def _body(labels_hbm, kb_hbm, out_hbm, labels_v, pairid_v, keys_v, out_v,
          *sems_all):
    sems, sem_out = sems_all[:_NG], sems_all[_NG]
    wid = lax.axis_index("s") * _NC + lax.axis_index("c")
    iota = lax.iota(jnp.int32, _L)
    cbase = pl.multiple_of(wid * _COLS, _COLS)

    pltpu.sync_copy(labels_hbm.at[:, pl.ds(cbase, _COLS)], labels_v)

    gathers = []
    for g in range(_NG):
        for cc in range(_CPG):
            c = g * _CPG + cc
            subj = labels_v[0, pl.ds(c * _L, _L)]
            obj = labels_v[1, pl.ds(c * _L, _L)]
            pairid_v[pl.ds(c * _L, _L)] = subj * _NUM_OBJ + obj
        gathers.append(
            pltpu.async_copy(
                kb_hbm.at[pairid_v.at[pl.ds(g * _GATHER_W, _GATHER_W)]],
                keys_v.at[pl.ds(g * _GATHER_W, _GATHER_W)],
                sems[g],
            )
        )

    zeros = jnp.zeros((_L,), jnp.float32)

    def _zero(j, carry):
        for b in range(_COLS // _L):
            out_v[j, pl.ds(b * _L, _L)] = zeros
        return carry

    lax.fori_loop(0, _NUM_RELS, _zero, 0)

    ones = jnp.full((_L,), 1.0, jnp.float32)
    out_copies = []
    for g in range(_NG):
        gathers[g].wait()
        for cc in range(_CPG):
            c = g * _CPG + cc
            keys = keys_v[pl.ds(c * _L, _L)]
            plsc.store_scatter(out_v, [keys, c * _L + iota], ones)
        out_copies.append(
            pltpu.async_copy(
                out_v.at[:, pl.ds(g * _GATHER_W, _GATHER_W)],
                out_hbm.at[:, pl.ds(cbase + g * _GATHER_W, _GATHER_W)],
                sem_out,
            )
        )
    for cp in out_copies:
        cp.wait()


@jax.jit
def _kb_bias_sc(labels_t, kb_table):
    mesh = plsc.VectorSubcoreMesh(core_axis_name="c", subcore_axis_name="s", num_cores=_NC)
    run = functools.partial(
        pl.kernel,
        out_type=jax.ShapeDtypeStruct((_NUM_RELS, _BATCH), jnp.float32),
        mesh=mesh,
        compiler_params=pltpu.CompilerParams(
            needs_layout_passes=False,
            skip_device_barrier=True,
            disable_bounds_checks=True,
            disable_semaphore_checks=True,
        ),
        scratch_types=[
            pltpu.VMEM((2, _COLS), jnp.int32),
            pltpu.VMEM((_COLS,), jnp.int32),
            pltpu.VMEM((_COLS,), jnp.int32),
            pltpu.VMEM((_NUM_RELS, _COLS), jnp.float32),
            *([pltpu.SemaphoreType.DMA] * _NG),
            pltpu.SemaphoreType.DMA,
        ],
    )(_body)
    return run(labels_t, kb_table)


def kernel(labels, kb_table):
    return _kb_bias_sc(labels.T, kb_table).T


# speedup vs baseline: 1.0584x; 1.0584x over previous
import functools

import jax
import jax.numpy as jnp
from jax import lax
from jax.experimental import pallas as pl
from jax.experimental.pallas import tpu as pltpu
from jax.experimental.pallas import tpu_sc as plsc

_NUM_OBJ = 151
_NUM_RELS = 51
_BATCH = 16384

_INFO = plsc.get_sparse_core_info()
_NC = _INFO.num_cores
_NS = _INFO.num_subcores
_NW = _NC * _NS
_L = _INFO.num_lanes
_COLS = _BATCH // _NW
_GATHER_W = 128
_NG = _COLS // _GATHER_W
_CPG = _GATHER_W // _L


def _body(labels_hbm, kb_hbm, out_hbm, labels_v, pairid_v, keys_v, out_v,
          sem_g0, sem_g1, sem_g2, sem_g3, sem_out):
    sems = [sem_g0, sem_g1, sem_g2, sem_g3]
    wid = lax.axis_index("s") * _NC + lax.axis_index("c")
    iota = lax.iota(jnp.int32, _L)
    cbase = pl.multiple_of(wid * _COLS, _COLS)

    pltpu.sync_copy(labels_hbm.at[:, pl.ds(cbase, _COLS)], labels_v)

    gathers = []
    for g in range(_NG):
        for cc in range(_CPG):
            c = g * _CPG + cc
            subj = labels_v[0, pl.ds(c * _L, _L)]
            obj = labels_v[1, pl.ds(c * _L, _L)]
            pairid_v[pl.ds(c * _L, _L)] = subj * _NUM_OBJ + obj
        gathers.append(
            pltpu.async_copy(
                kb_hbm.at[pairid_v.at[pl.ds(g * _GATHER_W, _GATHER_W)]],
                keys_v.at[pl.ds(g * _GATHER_W, _GATHER_W)],
                sems[g],
            )
        )

    zeros = jnp.zeros((_L,), jnp.float32)

    def _zero(j, carry):
        for b in range(_COLS // _L):
            out_v[j, pl.ds(b * _L, _L)] = zeros
        return carry

    lax.fori_loop(0, _NUM_RELS, _zero, 0)

    ones = jnp.full((_L,), 1.0, jnp.float32)
    out_copies = []
    for g in range(_NG):
        gathers[g].wait()
        for cc in range(_CPG):
            c = g * _CPG + cc
            keys = keys_v[pl.ds(c * _L, _L)]
            plsc.store_scatter(out_v, [keys, c * _L + iota], ones)
        out_copies.append(
            pltpu.async_copy(
                out_v.at[:, pl.ds(g * _GATHER_W, _GATHER_W)],
                out_hbm.at[:, pl.ds(cbase + g * _GATHER_W, _GATHER_W)],
                sem_out,
            )
        )
    for cp in out_copies:
        cp.wait()


@jax.jit
def _kb_bias_sc(labels_t, kb_table):
    mesh = plsc.VectorSubcoreMesh(core_axis_name="c", subcore_axis_name="s")
    run = functools.partial(
        pl.kernel,
        out_type=jax.ShapeDtypeStruct((_NUM_RELS, _BATCH), jnp.float32),
        mesh=mesh,
        compiler_params=pltpu.CompilerParams(
            needs_layout_passes=False,
            skip_device_barrier=True,
            disable_bounds_checks=True,
            disable_semaphore_checks=True,
        ),
        scratch_types=[
            pltpu.VMEM((2, _COLS), jnp.int32),
            pltpu.VMEM((_COLS,), jnp.int32),
            pltpu.VMEM((_COLS,), jnp.int32),
            pltpu.VMEM((_NUM_RELS, _COLS), jnp.float32),
            pltpu.SemaphoreType.DMA,
            pltpu.SemaphoreType.DMA,
            pltpu.SemaphoreType.DMA,
            pltpu.SemaphoreType.DMA,
            pltpu.SemaphoreType.DMA,
        ],
    )(_body)
    return run(labels_t, kb_table)


def kernel(labels, kb_table):
    return _kb_bias_sc(labels.T, kb_table).T


# speedup vs baseline: 1.0666x; 1.0078x over previous
import functools

import jax
import jax.numpy as jnp
from jax import lax
from jax.experimental import pallas as pl
from jax.experimental.pallas import tpu as pltpu
from jax.experimental.pallas import tpu_sc as plsc

_NUM_OBJ = 151
_NUM_RELS = 51
_BATCH = 16384

_INFO = plsc.get_sparse_core_info()
_NC = _INFO.num_cores
_NS = _INFO.num_subcores
_NW = _NC * _NS
_L = _INFO.num_lanes
_COLS = _BATCH // _NW
_GATHER_W = 128
_NG = _COLS // _GATHER_W
_CPG = _GATHER_W // _L


def _body(labels_hbm, kb_hbm, out_hbm, labels_v, pairid_v, keys_v, out_v,
          sem_g0, sem_g1, sem_g2, sem_g3, sem_out):
    sems = [sem_g0, sem_g1, sem_g2, sem_g3]
    wid = lax.axis_index("s") * _NC + lax.axis_index("c")
    iota = lax.iota(jnp.int32, _L)
    cbase = pl.multiple_of(wid * _COLS, _COLS)

    pltpu.sync_copy(labels_hbm.at[:, pl.ds(cbase, _COLS)], labels_v)

    def _pair(c, carry):
        off = pl.multiple_of(c * _L, _L)
        subj = labels_v[0, pl.ds(off, _L)]
        obj = labels_v[1, pl.ds(off, _L)]
        pairid_v[pl.ds(off, _L)] = subj * _NUM_OBJ + obj
        return carry

    gathers = []
    for g in range(_NG):
        lax.fori_loop(g * _CPG, (g + 1) * _CPG, _pair, 0)
        gathers.append(
            pltpu.async_copy(
                kb_hbm.at[pairid_v.at[pl.ds(g * _GATHER_W, _GATHER_W)]],
                keys_v.at[pl.ds(g * _GATHER_W, _GATHER_W)],
                sems[g],
            )
        )

    zeros = jnp.zeros((_L,), jnp.float32)

    def _zero(j, carry):
        for b in range(_COLS // _L):
            out_v[j, pl.ds(b * _L, _L)] = zeros
        return carry

    lax.fori_loop(0, _NUM_RELS, _zero, 0)

    ones = jnp.full((_L,), 1.0, jnp.float32)

    def _scat(c, carry):
        off = pl.multiple_of(c * _L, _L)
        keys = keys_v[pl.ds(off, _L)]
        plsc.store_scatter(out_v, [keys, off + iota], ones)
        return carry

    out_copies = []
    for g in range(_NG):
        gathers[g].wait()
        lax.fori_loop(g * _CPG, (g + 1) * _CPG, _scat, 0)
        out_copies.append(
            pltpu.async_copy(
                out_v.at[:, pl.ds(g * _GATHER_W, _GATHER_W)],
                out_hbm.at[:, pl.ds(cbase + g * _GATHER_W, _GATHER_W)],
                sem_out,
            )
        )
    for cp in out_copies:
        cp.wait()


@jax.jit
def _kb_bias_sc(labels_t, kb_table):
    mesh = plsc.VectorSubcoreMesh(core_axis_name="c", subcore_axis_name="s")
    run = functools.partial(
        pl.kernel,
        out_type=jax.ShapeDtypeStruct((_NUM_RELS, _BATCH), jnp.float32),
        mesh=mesh,
        compiler_params=pltpu.CompilerParams(
            needs_layout_passes=False,
            skip_device_barrier=True,
            disable_bounds_checks=True,
            disable_semaphore_checks=True,
        ),
        scratch_types=[
            pltpu.VMEM((2, _COLS), jnp.int32),
            pltpu.VMEM((_COLS,), jnp.int32),
            pltpu.VMEM((_COLS,), jnp.int32),
            pltpu.VMEM((_NUM_RELS, _COLS), jnp.float32),
            pltpu.SemaphoreType.DMA,
            pltpu.SemaphoreType.DMA,
            pltpu.SemaphoreType.DMA,
            pltpu.SemaphoreType.DMA,
            pltpu.SemaphoreType.DMA,
        ],
    )(_body)
    return run(labels_t, kb_table)


def kernel(labels, kb_table):
    return _kb_bias_sc(labels.T, kb_table).T
